# baseline (speedup 1.0000x reference)
import functools

import jax
import jax.numpy as jnp
from jax import lax
from jax.experimental import pallas as pl
from jax.experimental.pallas import tpu as pltpu
from jax.experimental.pallas import tpu_sc as plsc

NUM_EMBEDDINGS = 512
EMBEDDING_DIM = 32
COMMITMENT_COST = 0.25
BLOCK_TOKENS = 2048


def _vq_block_kernel(*refs, n_total, n_blocks, finalize):
    if finalize:
        (x_ref, cb_ref, pin_ref, idx_ref, com_ref, cbl_ref, tot_ref,
         acc_ref, c2_ref, cbt_ref) = refs
    else:
        (x_ref, cb_ref, idx_ref, pout_ref,
         acc_ref, c2_ref, cbt_ref) = refs
    i = pl.program_id(0)
    x = x_ref[...]
    cb = cb_ref[...]

    def _rowsum32(s):
        t = s[:, 0:8] + s[:, 8:16]
        t = t + s[:, 16:24]
        t = t + s[:, 24:32]
        t = t[:, 0:4] + t[:, 4:8]
        t = t[:, 0:2] + t[:, 2:4]
        return t[:, 0:1] + t[:, 1:2]

    @pl.when(i == 0)
    def _():
        c2_ref[...] = _rowsum32(cb * cb).reshape(1, -1)
        cbt_ref[...] = cb.astype(jnp.bfloat16).T

    x2 = _rowsum32(x * x)
    c2 = c2_ref[...]
    xc2 = jax.lax.dot_general(
        (x + x).astype(jnp.bfloat16), cbt_ref[...],
        (((1,), (0,)), ((), ())),
        preferred_element_type=jnp.float32)
    d2 = jnp.clip(x2 - xc2 + c2, 0.0, None)
    dist = jnp.sqrt(d2)
    b, k = d2.shape
    min_dist = jnp.min(dist, axis=1, keepdims=True)
    iota = jax.lax.broadcasted_iota(jnp.int32, (b, k), 1)
    idx = jnp.min(jnp.where(dist == min_dist, iota, k), axis=1)
    idx = idx.astype(jnp.int32)
    onehot = (iota == idx[:, None]).astype(jnp.float32)
    q = jax.lax.dot_general(
        onehot, cb, (((1,), (0,)), ((), ())),
        preferred_element_type=jnp.float32)
    idx_ref[...] = idx.reshape(1, 1, b)
    diff = x - q
    part = jnp.sum(diff * diff)

    @pl.when(i == 0)
    def _():
        acc_ref[0, 0] = part

    @pl.when(i > 0)
    def _():
        acc_ref[0, 0] += part

    @pl.when(i == n_blocks - 1)
    def _():
        if finalize:
            m = (acc_ref[0, 0] + pin_ref[0, 0]) / n_total
            com_ref[...] = jnp.full((1, 1), m * COMMITMENT_COST,
                                    jnp.float32)
            cbl_ref[...] = jnp.full((1, 1), m, jnp.float32)
            tot_ref[...] = jnp.full((1, 1), m * (1.0 + COMMITMENT_COST),
                                    jnp.float32)
        else:
            pout_ref[...] = jnp.full((1, 1), acc_ref[0, 0], jnp.float32)


def _tc_half(x_half, codebook, partial_in, *, n_total):
    n, d = x_half.shape
    k = codebook.shape[0]
    b = BLOCK_TOKENS
    n_blocks = n // b
    finalize = partial_in is not None
    body = functools.partial(_vq_block_kernel, n_total=n_total,
                             n_blocks=n_blocks, finalize=finalize)
    scalar_spec = pl.BlockSpec((1, 1), lambda i: (0, 0))
    in_specs = [
        pl.BlockSpec((b, d), lambda i: (i, 0)),
        pl.BlockSpec((k, d), lambda i: (0, 0)),
    ]
    operands = [x_half, codebook]
    if finalize:
        in_specs.append(scalar_spec)
        operands.append(partial_in)
        out_specs = [pl.BlockSpec((1, 1, b), lambda i: (i, 0, 0)),
                     scalar_spec, scalar_spec, scalar_spec]
        out_shape = [jax.ShapeDtypeStruct((n_blocks, 1, b), jnp.int32),
                     jax.ShapeDtypeStruct((1, 1), jnp.float32),
                     jax.ShapeDtypeStruct((1, 1), jnp.float32),
                     jax.ShapeDtypeStruct((1, 1), jnp.float32)]
    else:
        out_specs = [pl.BlockSpec((1, 1, b), lambda i: (i, 0, 0)),
                     scalar_spec]
        out_shape = [jax.ShapeDtypeStruct((n_blocks, 1, b), jnp.int32),
                     jax.ShapeDtypeStruct((1, 1), jnp.float32)]
    return pl.pallas_call(
        body,
        grid=(n_blocks,),
        in_specs=in_specs,
        out_specs=out_specs,
        out_shape=out_shape,
        scratch_shapes=[
            pltpu.SMEM((1, 1), jnp.float32),
            pltpu.VMEM((1, k), jnp.float32),
            pltpu.VMEM((d, k), jnp.bfloat16),
        ],
    )(*operands)


def _make_sc_gather(n, d, n_workers):
    b_per_w = n // n_workers
    mesh = plsc.VectorSubcoreMesh(core_axis_name="c", subcore_axis_name="s")

    @functools.partial(
        pl.kernel, mesh=mesh,
        out_type=jax.ShapeDtypeStruct((n, d), jnp.float32),
        compiler_params=pltpu.CompilerParams(use_tc_tiling_on_sc=False),
        scratch_types=[
            pltpu.VMEM((b_per_w,), jnp.int32),
            pltpu.VMEM((b_per_w, d), jnp.float32),
            pltpu.SemaphoreType.DMA,
        ],
    )
    def sc_gather(cb_hbm, idx_hbm, out_hbm, idx_v, rows_v, sem):
        wid = lax.axis_index("s") * 2 + lax.axis_index("c")
        base = wid * b_per_w
        pltpu.sync_copy(idx_hbm.at[pl.ds(base, b_per_w)], idx_v)
        pltpu.async_copy(cb_hbm.at[idx_v], rows_v, sem).wait()
        pltpu.sync_copy(rows_v, out_hbm.at[pl.ds(base, b_per_w)])

    return sc_gather


def kernel(z, codebook):
    orig_shape = z.shape
    d = codebook.shape[1]
    x = z.reshape(-1, d)
    n = x.shape[0]
    n_total = float(n * d)
    half = n // 2

    idx3_a, part_a = _tc_half(x[:half], codebook, None, n_total=n_total)
    idx_a = idx3_a.reshape(half)
    gather = _make_sc_gather(half, d, 32)
    q_a = gather(codebook, idx_a)

    idx3_b, com, cbl, tot = _tc_half(x[half:], codebook, part_a,
                                     n_total=n_total)
    idx_b = idx3_b.reshape(half)
    q_b = gather(codebook, idx_b)

    quantized = jnp.concatenate([q_a, q_b], axis=0).reshape(orig_shape)
    indices = jnp.concatenate([idx_a, idx_b], axis=0)
    return (quantized, indices, com.reshape(()), cbl.reshape(()),
            tot.reshape(()))


# speedup vs baseline: 1.1600x; 1.1600x over previous
import functools

import jax
import jax.numpy as jnp
from jax import lax
from jax.experimental import pallas as pl
from jax.experimental.pallas import tpu as pltpu
from jax.experimental.pallas import tpu_sc as plsc

NUM_EMBEDDINGS = 512
EMBEDDING_DIM = 32
COMMITMENT_COST = 0.25
BLOCK_TOKENS = 4096


def _vq_block_kernel(x_ref, cb_ref, idx_ref, com_ref, cbl_ref, tot_ref,
                     acc_ref, c2_ref, cbt_ref, *, n_total, n_blocks):
    i = pl.program_id(0)
    x = x_ref[...]
    cb = cb_ref[...]

    def _rowsum32(s):
        t = s[:, 0:8] + s[:, 8:16]
        t = t + s[:, 16:24]
        t = t + s[:, 24:32]
        t = t[:, 0:4] + t[:, 4:8]
        t = t[:, 0:2] + t[:, 2:4]
        return t[:, 0:1] + t[:, 1:2]

    @pl.when(i == 0)
    def _():
        c2_ref[...] = _rowsum32(cb * cb).reshape(1, -1)
        cbt_ref[...] = cb.astype(jnp.bfloat16).T

    x2 = _rowsum32(x * x)
    c2 = c2_ref[...]
    xc2 = jax.lax.dot_general(
        (x + x).astype(jnp.bfloat16), cbt_ref[...],
        (((1,), (0,)), ((), ())),
        preferred_element_type=jnp.float32)
    d2 = jnp.clip(x2 - xc2 + c2, 0.0, None)
    dist = jnp.sqrt(d2)
    b, k = d2.shape
    min_dist = jnp.min(dist, axis=1, keepdims=True)
    iota = jax.lax.broadcasted_iota(jnp.int32, (b, k), 1)
    idx = jnp.min(jnp.where(dist == min_dist, iota, k), axis=1)
    idx = idx.astype(jnp.int32)
    onehot = (iota == idx[:, None]).astype(jnp.float32)
    q = jax.lax.dot_general(
        onehot, cb, (((1,), (0,)), ((), ())),
        preferred_element_type=jnp.float32)
    idx_ref[...] = idx.reshape(1, 1, b)
    diff = x - q
    part = jnp.sum(diff * diff)

    @pl.when(i == 0)
    def _():
        acc_ref[0, 0] = part

    @pl.when(i > 0)
    def _():
        acc_ref[0, 0] += part

    @pl.when(i == n_blocks - 1)
    def _():
        m = acc_ref[0, 0] / n_total
        com_ref[...] = jnp.full((1, 1), m * COMMITMENT_COST, jnp.float32)
        cbl_ref[...] = jnp.full((1, 1), m, jnp.float32)
        tot_ref[...] = jnp.full((1, 1), m * (1.0 + COMMITMENT_COST),
                                jnp.float32)


def _make_sc_gather(n, d, n_workers):
    b_per_w = n // n_workers
    mesh = plsc.VectorSubcoreMesh(core_axis_name="c", subcore_axis_name="s")

    @functools.partial(
        pl.kernel, mesh=mesh,
        out_type=jax.ShapeDtypeStruct((n, d), jnp.float32),
        compiler_params=pltpu.CompilerParams(use_tc_tiling_on_sc=False),
        scratch_types=[
            pltpu.VMEM((b_per_w,), jnp.int32),
            pltpu.VMEM((b_per_w, d), jnp.float32),
            pltpu.SemaphoreType.DMA,
        ],
    )
    def sc_gather(cb_hbm, idx_hbm, out_hbm, idx_v, rows_v, sem):
        wid = lax.axis_index("s") * 2 + lax.axis_index("c")
        base = wid * b_per_w
        pltpu.sync_copy(idx_hbm.at[pl.ds(base, b_per_w)], idx_v)
        pltpu.async_copy(cb_hbm.at[idx_v], rows_v, sem).wait()
        pltpu.sync_copy(rows_v, out_hbm.at[pl.ds(base, b_per_w)])

    return sc_gather


def kernel(z, codebook):
    orig_shape = z.shape
    d = codebook.shape[1]
    x = z.reshape(-1, d)
    n = x.shape[0]
    b = BLOCK_TOKENS
    n_blocks = n // b
    k = codebook.shape[0]
    n_total = float(n * d)

    body = functools.partial(_vq_block_kernel, n_total=n_total,
                             n_blocks=n_blocks)
    scalar_spec = pl.BlockSpec((1, 1), lambda i: (0, 0))
    idx3, com, cbl, tot = pl.pallas_call(
        body,
        grid=(n_blocks,),
        in_specs=[
            pl.BlockSpec((b, d), lambda i: (i, 0)),
            pl.BlockSpec((k, d), lambda i: (0, 0)),
        ],
        out_specs=[
            pl.BlockSpec((1, 1, b), lambda i: (i, 0, 0)),
            scalar_spec, scalar_spec, scalar_spec,
        ],
        out_shape=[
            jax.ShapeDtypeStruct((n_blocks, 1, b), jnp.int32),
            jax.ShapeDtypeStruct((1, 1), jnp.float32),
            jax.ShapeDtypeStruct((1, 1), jnp.float32),
            jax.ShapeDtypeStruct((1, 1), jnp.float32),
        ],
        scratch_shapes=[
            pltpu.SMEM((1, 1), jnp.float32),
            pltpu.VMEM((1, k), jnp.float32),
            pltpu.VMEM((d, k), jnp.bfloat16),
        ],
    )(x, codebook)

    indices = idx3.reshape(n)
    q = _make_sc_gather(n, d, 32)(codebook, indices)
    quantized = q.reshape(orig_shape)
    return (quantized, indices, com.reshape(()), cbl.reshape(()),
            tot.reshape(()))


# speedup vs baseline: 1.1621x; 1.0018x over previous
import functools

import jax
import jax.numpy as jnp
from jax import lax
from jax.experimental import pallas as pl
from jax.experimental.pallas import tpu as pltpu
from jax.experimental.pallas import tpu_sc as plsc

NUM_EMBEDDINGS = 512
EMBEDDING_DIM = 32
COMMITMENT_COST = 0.25
BLOCK_TOKENS = 8192


def _vq_block_kernel(x_ref, cb_ref, idx_ref, com_ref, cbl_ref, tot_ref,
                     acc_ref, c2_ref, cbt_ref, *, n_total, n_blocks):
    i = pl.program_id(0)
    x = x_ref[...]
    cb = cb_ref[...]

    def _rowsum32(s):
        t = s[:, 0:8] + s[:, 8:16]
        t = t + s[:, 16:24]
        t = t + s[:, 24:32]
        t = t[:, 0:4] + t[:, 4:8]
        t = t[:, 0:2] + t[:, 2:4]
        return t[:, 0:1] + t[:, 1:2]

    @pl.when(i == 0)
    def _():
        c2_ref[...] = _rowsum32(cb * cb).reshape(1, -1)
        cbt_ref[...] = cb.astype(jnp.bfloat16).T

    x2 = _rowsum32(x * x)
    c2 = c2_ref[...]
    xc2 = jax.lax.dot_general(
        (x + x).astype(jnp.bfloat16), cbt_ref[...],
        (((1,), (0,)), ((), ())),
        preferred_element_type=jnp.float32)
    d2 = jnp.clip(x2 - xc2 + c2, 0.0, None)
    dist = jnp.sqrt(d2)
    b, k = d2.shape
    min_dist = jnp.min(dist, axis=1, keepdims=True)
    iota = jax.lax.broadcasted_iota(jnp.int32, (b, k), 1)
    idx = jnp.min(jnp.where(dist == min_dist, iota, k), axis=1)
    idx = idx.astype(jnp.int32)
    onehot = (iota == idx[:, None]).astype(jnp.float32)
    q = jax.lax.dot_general(
        onehot, cb, (((1,), (0,)), ((), ())),
        preferred_element_type=jnp.float32)
    idx_ref[...] = idx.reshape(1, 1, b)
    diff = x - q
    part = jnp.sum(diff * diff)

    @pl.when(i == 0)
    def _():
        acc_ref[0, 0] = part

    @pl.when(i > 0)
    def _():
        acc_ref[0, 0] += part

    @pl.when(i == n_blocks - 1)
    def _():
        m = acc_ref[0, 0] / n_total
        com_ref[...] = jnp.full((1, 1), m * COMMITMENT_COST, jnp.float32)
        cbl_ref[...] = jnp.full((1, 1), m, jnp.float32)
        tot_ref[...] = jnp.full((1, 1), m * (1.0 + COMMITMENT_COST),
                                jnp.float32)


def _make_sc_gather(n, d, n_workers):
    b_per_w = n // n_workers
    mesh = plsc.VectorSubcoreMesh(core_axis_name="c", subcore_axis_name="s")

    @functools.partial(
        pl.kernel, mesh=mesh,
        out_type=jax.ShapeDtypeStruct((n, d), jnp.float32),
        compiler_params=pltpu.CompilerParams(use_tc_tiling_on_sc=False),
        scratch_types=[
            pltpu.VMEM((b_per_w,), jnp.int32),
            pltpu.VMEM((b_per_w, d), jnp.float32),
            pltpu.SemaphoreType.DMA,
        ],
    )
    def sc_gather(cb_hbm, idx_hbm, out_hbm, idx_v, rows_v, sem):
        wid = lax.axis_index("s") * 2 + lax.axis_index("c")
        base = wid * b_per_w
        pltpu.sync_copy(idx_hbm.at[pl.ds(base, b_per_w)], idx_v)
        pltpu.async_copy(cb_hbm.at[idx_v], rows_v, sem).wait()
        pltpu.sync_copy(rows_v, out_hbm.at[pl.ds(base, b_per_w)])

    return sc_gather


def kernel(z, codebook):
    orig_shape = z.shape
    d = codebook.shape[1]
    x = z.reshape(-1, d)
    n = x.shape[0]
    b = BLOCK_TOKENS
    n_blocks = n // b
    k = codebook.shape[0]
    n_total = float(n * d)

    body = functools.partial(_vq_block_kernel, n_total=n_total,
                             n_blocks=n_blocks)
    scalar_spec = pl.BlockSpec((1, 1), lambda i: (0, 0))
    idx3, com, cbl, tot = pl.pallas_call(
        body,
        grid=(n_blocks,),
        in_specs=[
            pl.BlockSpec((b, d), lambda i: (i, 0)),
            pl.BlockSpec((k, d), lambda i: (0, 0)),
        ],
        out_specs=[
            pl.BlockSpec((1, 1, b), lambda i: (i, 0, 0)),
            scalar_spec, scalar_spec, scalar_spec,
        ],
        out_shape=[
            jax.ShapeDtypeStruct((n_blocks, 1, b), jnp.int32),
            jax.ShapeDtypeStruct((1, 1), jnp.float32),
            jax.ShapeDtypeStruct((1, 1), jnp.float32),
            jax.ShapeDtypeStruct((1, 1), jnp.float32),
        ],
        scratch_shapes=[
            pltpu.SMEM((1, 1), jnp.float32),
            pltpu.VMEM((1, k), jnp.float32),
            pltpu.VMEM((d, k), jnp.bfloat16),
        ],
    )(x, codebook)

    indices = idx3.reshape(n)
    q = _make_sc_gather(n, d, 32)(codebook, indices)
    quantized = q.reshape(orig_shape)
    return (quantized, indices, com.reshape(()), cbl.reshape(()),
            tot.reshape(()))
